# baseline (speedup 1.0000x reference)
import functools

import jax
import jax.numpy as jnp
from jax import lax
from jax.experimental import pallas as pl
from jax.experimental.pallas import tpu as pltpu
from jax.experimental.pallas import tpu_sc as plsc

_IN_DIM = 100
_SUB = 9997
_DENSITY = 10000
_OUT = 64
_B = 1024
_R = 4 * _IN_DIM
_NW = 32
_NB = _B // _NW
_NCHUNK = 5
_CW = _R // _NCHUNK


def _prep_body(x_ref, idx_ref, w_ref):
    x = x_ref[:] * float(_SUB)
    xf = jnp.floor(x)
    t = x - xf
    t2 = t * t
    t3 = t2 * t
    p1 = t3 / 6.0
    p2 = (-3.0 * t3 + 3.0 * t2 + 3.0 * t + 1.0) / 6.0
    p3 = (3.0 * t3 - 6.0 * t2 + 4.0) / 6.0
    omt = 1.0 - t
    p4 = omt * omt * omt / 6.0
    shift = (lax.broadcasted_iota(jnp.int32, x.shape, 1) * _DENSITY
             ).astype(jnp.float32)
    idxs = [
        (jnp.mod(xf + float(k), float(_DENSITY)) + shift).astype(jnp.int32)
        for k in range(4)
    ]
    idx_ref[:] = jnp.concatenate(idxs, axis=1)
    w_ref[:] = jnp.concatenate([p4, p3, p2, p1], axis=1)


_prep = pl.pallas_call(
    _prep_body,
    out_shape=(
        jax.ShapeDtypeStruct((_B, _R), jnp.int32),
        jax.ShapeDtypeStruct((_B, _R), jnp.float32),
    ),
)


def _splat(v, i):
    idx = jnp.full((16, 1), i, jnp.int32)
    dn = lax.GatherDimensionNumbers(
        offset_dims=(), collapsed_slice_dims=(0,), start_index_map=(0,))
    return lax.gather(v, idx, dn, (1,),
                      mode=lax.GatherScatterMode.PROMISE_IN_BOUNDS)


@functools.cache
def _build_sc():
    mesh = plsc.VectorSubcoreMesh(core_axis_name="c", subcore_axis_name="s")

    @functools.partial(
        pl.kernel,
        out_type=jax.ShapeDtypeStruct((_B, _OUT), jnp.float32),
        mesh=mesh,
        scratch_types=[
            pltpu.VMEM((2, _NCHUNK, _CW), jnp.int32),
            pltpu.VMEM((2, _R), jnp.float32),
            pltpu.VMEM((2, _NCHUNK, _CW, _OUT), jnp.float32),
            pltpu.VMEM((2, _OUT), jnp.float32),
            pltpu.SemaphoreType.DMA,
            pltpu.SemaphoreType.DMA,
        ],
        compiler_params=pltpu.CompilerParams(use_tc_tiling_on_sc=False),
    )
    def _sc_gather(table_hbm, idx_hbm, w_hbm, out_hbm,
                   idx_v, w_v, rows_v, orow_v, sem0, sem1):
        wid = lax.axis_index("s") * 2 + lax.axis_index("c")
        base = wid * _NB
        sems = (sem0, sem1)

        def issue(bb, buf):
            pltpu.sync_copy(idx_hbm.at[bb], idx_v.at[buf])
            pltpu.sync_copy(w_hbm.at[bb], w_v.at[buf])
            for j in range(_NCHUNK):
                pltpu.async_copy(
                    table_hbm.at[idx_v.at[buf, j]], rows_v.at[buf, j],
                    sems[buf])

        def drain(buf):
            for j in range(_NCHUNK):
                pltpu.make_async_copy(
                    table_hbm.at[idx_v.at[buf, j]], rows_v.at[buf, j],
                    sems[buf]).wait()

        def consume(bb, buf):
            drain(buf)
            accs = tuple(jnp.zeros((16,), jnp.float32) for _ in range(4))
            for j in range(_NCHUNK):
                def chunk_body(sub, accs, j=j):
                    a0, a1, a2, a3 = accs
                    w16 = w_v[buf, pl.ds(j * _CW + sub * 16, 16)]
                    for i in range(16):
                        r = sub * 16 + i
                        wi = _splat(w16, i)
                        a0 = a0 + wi * rows_v[buf, j, r, pl.ds(0, 16)]
                        a1 = a1 + wi * rows_v[buf, j, r, pl.ds(16, 16)]
                        a2 = a2 + wi * rows_v[buf, j, r, pl.ds(32, 16)]
                        a3 = a3 + wi * rows_v[buf, j, r, pl.ds(48, 16)]
                    return (a0, a1, a2, a3)
                accs = lax.fori_loop(0, _CW // 16, chunk_body, accs)
            for q in range(4):
                orow_v[buf, pl.ds(q * 16, 16)] = accs[q]
            pltpu.sync_copy(orow_v.at[buf], out_hbm.at[bb])

        issue(base, 0)
        issue(base + 1, 1)

        def step(tt, carry):
            b0 = base + 2 * tt
            consume(b0, 0)

            @pl.when(tt < _NB // 2 - 1)
            def _():
                issue(b0 + 2, 0)

            consume(b0 + 1, 1)

            @pl.when(tt < _NB // 2 - 1)
            def _():
                issue(b0 + 3, 1)

            return carry

        lax.fori_loop(0, _NB // 2, step, 0)

    return _sc_gather


def kernel(inputs, table):
    idx, w = _prep(inputs)
    idx3 = idx.reshape(_B, _NCHUNK, _CW)
    return _build_sc()(table, idx3, w)
